# baseline (speedup 1.0000x reference)
import functools

import jax
import jax.numpy as jnp
from jax import lax
from jax.experimental import pallas as pl
from jax.experimental.pallas import tpu as pltpu
from jax.experimental.pallas import tpu_sc as plsc

N = 50000
E = 400000
IN_C = 16
HID = 16

_NC = 2
_NS = 16
_NW = _NC * _NS
_GRP = 128
_G = E // _GRP
_GPW = 98
_GP = _GPW * _NW
_EP = _GP * _GRP
_BK = 14
_NBURST = _GPW // _BK
_BKS = 7
_NBURSTS = _GPW // _BKS
_STRIPE = N // _NS
_NPAD = N + 8


def _sc_gather(x, src_groups):
  mesh = plsc.VectorSubcoreMesh(core_axis_name="c", subcore_axis_name="s",
                                num_cores=_NC, num_subcores=_NS)

  @functools.partial(
      pl.kernel,
      out_type=jax.ShapeDtypeStruct((_GP, _GRP, IN_C), jnp.float32),
      mesh=mesh,
      scratch_types=[
          pltpu.VMEM((_BK, _GRP), jnp.int32),
          pltpu.VMEM((_BK, _GRP, IN_C), jnp.float32),
          pltpu.SemaphoreType.DMA,
      ],
      compiler_params=pltpu.CompilerParams(use_tc_tiling_on_sc=False),
  )
  def k(x_hbm, src_hbm, out_hbm, idx_v, rows_v, sem):
    wid = lax.axis_index("s") * _NC + lax.axis_index("c")
    base = wid * _GPW

    def body(t, carry):
      g0 = base + t * _BK
      pltpu.sync_copy(src_hbm.at[pl.ds(g0, _BK)], idx_v)
      descs = [pltpu.async_copy(x_hbm.at[idx_v.at[j]], rows_v.at[j], sem)
               for j in range(_BK)]
      for d in descs:
        d.wait()
      pltpu.sync_copy(rows_v, out_hbm.at[pl.ds(g0, _BK)])
      return carry

    lax.fori_loop(0, _NBURST, body, 0)

  return k(x, src_groups)


def _sc_scatter(comb, dst_groups, zeros_init):
  mesh = plsc.VectorSubcoreMesh(core_axis_name="c", subcore_axis_name="s",
                                num_cores=_NC, num_subcores=_NS)

  @functools.partial(
      pl.kernel,
      out_type=jax.ShapeDtypeStruct((_NC * N, 2 * HID), jnp.float32),
      mesh=mesh,
      scratch_types=[
          pltpu.VMEM_SHARED((_NPAD, 2 * HID), jnp.float32),
          pltpu.VMEM((_BKS, _GRP), jnp.int32),
          pltpu.VMEM((_BKS * _GRP, 2 * HID), jnp.float32),
          pltpu.SemaphoreType.DMA,
      ],
      compiler_params=pltpu.CompilerParams(use_tc_tiling_on_sc=False),
  )
  def k(comb_hbm, dst_hbm, zeros_hbm, out_hbm, acc, dst_v, comb_v, sem):
    cid = lax.axis_index("c")
    sid = lax.axis_index("s")
    wid = sid * _NC + cid
    base = wid * _GPW

    pltpu.sync_copy(zeros_hbm.at[pl.ds(sid * _STRIPE, _STRIPE)],
                    acc.at[pl.ds(sid * _STRIPE, _STRIPE)])
    plsc.subcore_barrier()

    def body(t, carry):
      g0 = base + t * _BKS
      pltpu.sync_copy(dst_hbm.at[pl.ds(g0, _BKS)], dst_v)
      pltpu.sync_copy(comb_hbm.at[pl.ds(g0 * _GRP, _BKS * _GRP)], comb_v)
      descs = [
          pltpu.async_copy(comb_v.at[pl.ds(j * _GRP, _GRP)],
                           acc.at[dst_v.at[j]], sem, add=True)
          for j in range(_BKS)
      ]
      for d in descs:
        d.wait()
      return carry

    lax.fori_loop(0, _NBURSTS, body, 0)
    plsc.subcore_barrier()

    pltpu.sync_copy(acc.at[pl.ds(sid * _STRIPE, _STRIPE)],
                    out_hbm.at[pl.ds(cid * N + sid * _STRIPE, _STRIPE)])

  return k(comb, dst_groups, zeros_init)


_BE = 3200
_RE = _BE // 8


def _edge_body(ea_ref, xg_ref, b1p_ref, b2p_ref, w1p_ref, w2p_ref, rp_ref,
               spc_ref, ones_ref, comb_ref):
  ea = ea_ref[...]
  eh = jnp.maximum(
      jnp.dot(ea, w1p_ref[...], preferred_element_type=jnp.float32)
      + b1p_ref[...], 0.0)
  we = jnp.dot(eh, w2p_ref[...],
               preferred_element_type=jnp.float32) + b2p_ref[...]
  xrep = jnp.dot(xg_ref[...], rp_ref[...],
                 preferred_element_type=jnp.float32)
  prod = we * xrep
  comb_ref[...] = jnp.dot(prod, spc_ref[...],
                          preferred_element_type=jnp.float32) + ones_ref[...]


def _tc_edge_mlp(eap8, xgp, b1p, b2p, w1p, w2p, rp, spc, onesmask):
  grid = (E // _BE,)
  return pl.pallas_call(
      _edge_body,
      grid=grid,
      in_specs=[
          pl.BlockSpec((_RE, 32), lambda i: (i, 0)),
          pl.BlockSpec((_RE, 128), lambda i: (i, 0)),
          pl.BlockSpec((1, 256), lambda i: (0, 0)),
          pl.BlockSpec((1, 2048), lambda i: (0, 0)),
          pl.BlockSpec((32, 256), lambda i: (0, 0)),
          pl.BlockSpec((256, 2048), lambda i: (0, 0)),
          pl.BlockSpec((128, 2048), lambda i: (0, 0)),
          pl.BlockSpec((2048, 256), lambda i: (0, 0)),
          pl.BlockSpec((1, 256), lambda i: (0, 0)),
      ],
      out_specs=pl.BlockSpec((_RE, 256), lambda i: (i, 0)),
      out_shape=jax.ShapeDtypeStruct((_EP // 8, 256), jnp.float32),
  )(eap8, xgp, b1p, b2p, w1p, w2p, rp, spc, onesmask)


_BN = 2000


def _dense_body(x_ref, hp1_ref, hp2_ref, c0_ref, c1_ref, root_ref, rb_ref,
                u_ref, tb_ref, woutt_ref, bo_ref, hout_ref, hg_ref):
  c0 = c0_ref[...]
  c1 = c1_ref[...]
  agg = c0[:, :16] + c1[:, :16]
  cnt = c0[:, 16:] + c1[:, 16:]
  mean = agg / jnp.maximum(cnt, 1.0)
  hg = jnp.maximum(
      mean + jnp.dot(x_ref[...], root_ref[...],
                     preferred_element_type=jnp.float32) + rb_ref[...], 0.0)
  hcat = jnp.concatenate([hp1_ref[...], hp2_ref[...], hg], axis=1)
  outcat = jnp.dot(hcat, u_ref[...], preferred_element_type=jnp.float32) \
      + tb_ref[...]
  hout_ref[...] = jnp.dot(outcat, woutt_ref[...],
                          preferred_element_type=jnp.float32) + bo_ref[...]
  hg_ref[...] = hg


def _tc_dense(x, hp1, hp2, comb2, root, rbr, u, tbcat, woutt, boutr):
  grid = (N // _BN,)
  nblk = N // _BN
  return pl.pallas_call(
      _dense_body,
      grid=grid,
      in_specs=[
          pl.BlockSpec((_BN, IN_C), lambda i: (i, 0)),
          pl.BlockSpec((_BN, HID), lambda i: (i, 0)),
          pl.BlockSpec((_BN, HID), lambda i: (i, 0)),
          pl.BlockSpec((_BN, 2 * HID), lambda i: (i, 0)),
          pl.BlockSpec((_BN, 2 * HID), lambda i, n=nblk: (i + n, 0)),
          pl.BlockSpec((IN_C, HID), lambda i: (0, 0)),
          pl.BlockSpec((1, HID), lambda i: (0, 0)),
          pl.BlockSpec((48, 48), lambda i: (0, 0)),
          pl.BlockSpec((1, 48), lambda i: (0, 0)),
          pl.BlockSpec((48, HID), lambda i: (0, 0)),
          pl.BlockSpec((1, HID), lambda i: (0, 0)),
      ],
      out_specs=[
          pl.BlockSpec((_BN, HID), lambda i: (i, 0)),
          pl.BlockSpec((_BN, HID), lambda i: (i, 0)),
      ],
      out_shape=[
          jax.ShapeDtypeStruct((N, HID), jnp.float32),
          jax.ShapeDtypeStruct((N, HID), jnp.float32),
      ],
  )(x, hp1, hp2, comb2, comb2, root, rbr, u, tbcat, woutt, boutr)


def _packed_weights(W1, b1, W2, b2):
  f32 = jnp.float32
  r32 = jnp.arange(32)
  r128 = jnp.arange(128)
  r256 = jnp.arange(256)
  r2048 = jnp.arange(2048)
  w1p = jnp.where((r32[:, None] // 4 == r256[None, :] // 32),
                  W1[r256[None, :] % 32, r32[:, None] % 4], 0.0).astype(f32)
  b1p = jnp.tile(b1, 8)[None, :]
  w2p = jnp.where((r256[:, None] // 32 == r2048[None, :] // 256),
                  W2[r2048[None, :] % 256, r256[:, None] % 32], 0.0).astype(f32)
  b2p = jnp.tile(b2, 8)[None, :]
  rp = ((r128[:, None] // 16 == r2048[None, :] // 256)
        & (r128[:, None] % 16 == (r2048[None, :] % 256) // 16)).astype(f32)
  spc = ((r2048[:, None] // 256 == r256[None, :] // 32)
         & (r2048[:, None] % 16 == r256[None, :] % 32)).astype(f32)
  onesmask = (r256 % 32 >= 16).astype(f32)[None, :]
  return w1p, b1p, w2p, b2p, rp, spc, onesmask


def kernel(x, edge_index, edge_attr, h_prev, W1, b1, W2, b2, root, root_bias,
           tw1, tb1, tw2, tb2, tw3, tb3, Wout, bout):
  src_groups = jnp.pad(edge_index[0].reshape(_G, _GRP), ((0, _GP - _G), (0, 0)))
  dst_groups = jnp.pad(edge_index[1].reshape(_G, _GRP), ((0, _GP - _G), (0, 0)),
                       constant_values=N)

  xg3 = _sc_gather(x, src_groups)
  xgp = xg3.reshape(_EP * IN_C // 128, 128)

  w1p, b1p, w2p, b2p, rp, spc, onesmask = _packed_weights(W1, b1, W2, b2)
  eap8 = edge_attr.reshape(E // 8, 32)
  combp = _tc_edge_mlp(eap8, xgp, b1p, b2p, w1p, w2p, rp, spc, onesmask)
  comb = combp.reshape(_EP, 2 * HID)

  zeros_init = jnp.zeros((_NPAD, 2 * HID), jnp.float32)
  combparts = _sc_scatter(comb, dst_groups, zeros_init)

  t = jnp.zeros((48, 48), jnp.float32)
  t = t.at[0:16, 16:32].set(tw2[:, :, 0].T)
  t = t.at[16:32, 0:16].set(tw1[:, :, 0].T)
  t = t.at[32:48, 0:16].set(tw1[:, :, 1].T)
  t = t.at[32:48, 16:32].set(tw2[:, :, 1].T)
  t = t.at[32:48, 32:48].set(tw3[:, :, 1].T)
  tbcat = jnp.concatenate([tb1, tb2, tb3])[None, :]
  hp1 = h_prev[:, 1, :]
  hp2 = h_prev[:, 2, :]
  h_out, hg = _tc_dense(x, hp1, hp2, combparts, root, root_bias[None, :],
                        t, tbcat, Wout.T, bout[None, :])
  h_hist = jnp.concatenate([h_prev[:, 1:], hg[:, None, :]], axis=1)
  return h_out, h_hist
